# baseline (speedup 1.0000x reference)
import jax
import jax.numpy as jnp
from jax.experimental import pallas as pl
from jax.experimental.pallas import tpu as pltpu
from jax.experimental.pallas import tpu_sc as plsc

_N = 256
_FEATS = 768
_BATCH = 32

_PJ = {1: 32, 2: 64, 4: 128, 8: 8, 16: 16, 32: 1, 64: 2, 128: 4}
_DK = {2: 64, 4: 128, 8: 8, 16: 16, 32: 1, 64: 2, 128: 4}


def _stage(a, k, j):
    n, cols = a.shape
    pj = _PJ[j]
    dk = _DK.get(k)
    if pj >= 8:
        g = n // (2 * pj)
        a4 = a.reshape(g, 2, pj, cols)
        mn = jnp.minimum(a4[:, 0], a4[:, 1]).reshape(n // 2, cols)
        mx = jnp.maximum(a4[:, 0], a4[:, 1]).reshape(n // 2, cols)
        if dk is None:
            nl, nh = mx, mn
        else:
            d = dk // 2 if dk >= 2 * pj else dk
            if d >= 8:
                m7 = mn.reshape(n // (4 * d), 2, d, cols)
                x7 = mx.reshape(n // (4 * d), 2, d, cols)
                nl = jnp.concatenate([x7[:, :1], m7[:, 1:]], axis=1)
                nl = nl.reshape(n // 2, cols)
                nh = jnp.concatenate([m7[:, :1], x7[:, 1:]], axis=1)
                nh = nh.reshape(n // 2, cols)
            else:
                q = jax.lax.broadcasted_iota(jnp.int32, (n // 2, cols), 0)
                ascm = (q & d) != 0
                nl = jnp.where(ascm, mn, mx)
                nh = jnp.where(ascm, mx, mn)
        return jnp.stack(
            [nl.reshape(g, pj, cols), nh.reshape(g, pj, cols)], axis=1
        ).reshape(n, cols)
    row = jax.lax.broadcasted_iota(jnp.int32, (n, cols), 0)
    bitp = (row & pj) != 0
    if pj == 4:
        p = jnp.roll(a.reshape(n // 8, 8, cols), 4, axis=1).reshape(n, cols)
    else:
        p = jnp.where(bitp, jnp.roll(a, pj, axis=0), jnp.roll(a, -pj, axis=0))
    if dk is None:
        take_min = bitp
    else:
        take_min = jnp.logical_xor((row & dk) != 0, bitp)
    return jnp.where(take_min, jnp.minimum(a, p), jnp.maximum(a, p))


_BPS = 2
_CW = 128
_NSC = 4


def _corr_sort_body(x_ref, o_ref):
    for b in range(_BPS):
        x = x_ref[b]
        for h in range(_N // _CW):
            cs = slice(h * _CW, (h + 1) * _CW)
            a = jax.lax.dot_general(
                x,
                x[:, cs],
                (((0,), (0,)), ((), ())),
                preferred_element_type=jnp.float32,
            ) * (1.0 / _FEATS)

            k = 2
            while k <= _N:
                j = k // 2
                while j >= 1:
                    a = _stage(a, k, j)
                    j //= 2
                k *= 2
            o_ref[b, :, cs] = (
                a.reshape(8, 4, 8, _CW).transpose(2, 1, 0, 3).reshape(_N, _CW)
            )


def _corr_only_body(x_ref, o_ref):
    for b in range(_NSC):
        x = x_ref[b]
        o_ref[b] = jax.lax.dot_general(
            x, x, (((0,), (0,)), ((), ())), preferred_element_type=jnp.float32
        ) * (1.0 / _FEATS)


def _sc_tile_body(c_vmem, o_vmem):
    def cmp_block(src, dst, base, j, desc):
        @pl.loop(0, j)
        def _(c):
            i = base + c
            lo = src[0, i]
            hi = src[0, i + j]
            if desc:
                dst[0, i] = jnp.maximum(lo, hi)
                dst[0, i + j] = jnp.minimum(lo, hi)
            else:
                dst[0, i] = jnp.minimum(lo, hi)
                dst[0, i + j] = jnp.maximum(lo, hi)

    first = True
    k = 2
    while k <= _N:
        j = k // 2
        while j >= 1:
            src = c_vmem if first else o_vmem
            if k == _N:

                @pl.loop(0, _N, step=2 * j)
                def _(b0):
                    cmp_block(src, o_vmem, b0, j, True)

            else:

                @pl.loop(0, _N, step=2 * k)
                def _(s):
                    @pl.loop(0, k, step=2 * j)
                    def _(o):
                        cmp_block(src, o_vmem, s + o, j, True)

                    @pl.loop(0, k, step=2 * j)
                    def _(o):
                        cmp_block(src, o_vmem, s + k + o, j, False)

            first = False
            j //= 2
        k *= 2


def _sc_sort(c):
    nsc = c.shape[0]

    @pl.kernel(
        out_type=jax.ShapeDtypeStruct(c.shape, c.dtype),
        mesh=plsc.VectorSubcoreMesh(core_axis_name="c", subcore_axis_name="s"),
        compiler_params=pltpu.CompilerParams(use_tc_tiling_on_sc=False),
    )
    def sc_fn(c_hbm, o_hbm):
        pltpu.emit_pipeline(
            _sc_tile_body,
            grid=(nsc, _N // 16),
            in_specs=[pl.BlockSpec((1, _N, 16), lambda b, t: (b, 0, t))],
            out_specs=[pl.BlockSpec((1, _N, 16), lambda b, t: (b, 0, t))],
            core_axis_name=("c", "s"),
            dimension_semantics=(pltpu.PARALLEL, pltpu.PARALLEL),
        )(c_hbm, o_hbm)

    return sc_fn(c)


def kernel(x):
    n_bsize, n_feats, n_cols, n_rows = x.shape
    x3 = x.reshape(n_bsize, n_feats, n_cols * n_rows)
    c_sc = pl.pallas_call(
        _corr_only_body,
        grid=(1,),
        in_specs=[pl.BlockSpec((_NSC, n_feats, _N), lambda b: (0, 0, 0))],
        out_specs=pl.BlockSpec((_NSC, _N, _N), lambda b: (0, 0, 0)),
        out_shape=jax.ShapeDtypeStruct((_NSC, _N, _N), jnp.float32),
    )(x3[:_NSC])
    out_sc = _sc_sort(c_sc)
    out_tc = pl.pallas_call(
        _corr_sort_body,
        grid=((n_bsize - _NSC) // _BPS,),
        in_specs=[pl.BlockSpec((_BPS, n_feats, _N), lambda b: (b, 0, 0))],
        out_specs=pl.BlockSpec((_BPS, _N, _N), lambda b: (b, 0, 0)),
        out_shape=jax.ShapeDtypeStruct((n_bsize - _NSC, _N, _N), jnp.float32),
    )(x3[_NSC:])
    out = jnp.concatenate([out_sc, out_tc], axis=0)
    return out.reshape(n_bsize, _N, n_cols, n_rows)


# speedup vs baseline: 1.1777x; 1.1777x over previous
import jax
import jax.numpy as jnp
from jax.experimental import pallas as pl
from jax.experimental.pallas import tpu as pltpu
from jax.experimental.pallas import tpu_sc as plsc

_N = 256
_FEATS = 768
_BATCH = 32

_PJ = {1: 32, 2: 64, 4: 128, 8: 8, 16: 16, 32: 1, 64: 2, 128: 4}
_DK = {2: 64, 4: 128, 8: 8, 16: 16, 32: 1, 64: 2, 128: 4}


def _stage(a, k, j):
    n, cols = a.shape
    pj = _PJ[j]
    dk = _DK.get(k)
    if pj >= 8:
        g = n // (2 * pj)
        a4 = a.reshape(g, 2, pj, cols)
        mn = jnp.minimum(a4[:, 0], a4[:, 1]).reshape(n // 2, cols)
        mx = jnp.maximum(a4[:, 0], a4[:, 1]).reshape(n // 2, cols)
        if dk is None:
            nl, nh = mx, mn
        else:
            d = dk // 2 if dk >= 2 * pj else dk
            if d >= 8:
                m7 = mn.reshape(n // (4 * d), 2, d, cols)
                x7 = mx.reshape(n // (4 * d), 2, d, cols)
                nl = jnp.concatenate([x7[:, :1], m7[:, 1:]], axis=1)
                nl = nl.reshape(n // 2, cols)
                nh = jnp.concatenate([m7[:, :1], x7[:, 1:]], axis=1)
                nh = nh.reshape(n // 2, cols)
            else:
                q = jax.lax.broadcasted_iota(jnp.int32, (n // 2, cols), 0)
                ascm = (q & d) != 0
                nl = jnp.where(ascm, mn, mx)
                nh = jnp.where(ascm, mx, mn)
        return jnp.stack(
            [nl.reshape(g, pj, cols), nh.reshape(g, pj, cols)], axis=1
        ).reshape(n, cols)
    row = jax.lax.broadcasted_iota(jnp.int32, (n, cols), 0)
    bitp = (row & pj) != 0
    if pj == 4:
        p = jnp.roll(a.reshape(n // 8, 8, cols), 4, axis=1).reshape(n, cols)
    else:
        p = jnp.where(bitp, jnp.roll(a, pj, axis=0), jnp.roll(a, -pj, axis=0))
    if dk is None:
        take_min = bitp
    else:
        take_min = jnp.logical_xor((row & dk) != 0, bitp)
    return jnp.where(take_min, jnp.minimum(a, p), jnp.maximum(a, p))


_BPS = 2
_CW = 128
_NSC = 4


def _corr_sort_body(x_ref, o_ref):
    for b in range(_BPS):
        x = x_ref[b]
        for h in range(_N // _CW):
            cs = slice(h * _CW, (h + 1) * _CW)
            a = jax.lax.dot_general(
                x,
                x[:, cs],
                (((0,), (0,)), ((), ())),
                preferred_element_type=jnp.float32,
            ) * (1.0 / _FEATS)

            k = 2
            while k <= _N:
                j = k // 2
                while j >= 1:
                    a = _stage(a, k, j)
                    j //= 2
                k *= 2
            o_ref[b, :, cs] = (
                a.reshape(8, 4, 8, _CW).transpose(2, 1, 0, 3).reshape(_N, _CW)
            )


def _corr_only_body(x_ref, o_ref):
    for b in range(_NSC):
        x = x_ref[b]
        o_ref[b] = jax.lax.dot_general(
            x, x, (((0,), (0,)), ((), ())), preferred_element_type=jnp.float32
        ) * (1.0 / _FEATS)


def _sc_tile_body(c_vmem, o_vmem):
    def cmp(src, dst, i, j, desc):
        lo = src[0, i]
        hi = src[0, i + j]
        if desc:
            dst[0, i] = jnp.maximum(lo, hi)
            dst[0, i + j] = jnp.minimum(lo, hi)
        else:
            dst[0, i] = jnp.minimum(lo, hi)
            dst[0, i + j] = jnp.maximum(lo, hi)

    first = True
    k = 2
    while k <= _N:
        j = k // 2
        while j >= 1:
            src = c_vmem if first else o_vmem
            jm = j - 1
            if k == _N:

                @plsc.parallel_loop(0, _N // 2, unroll=4)
                def _(p):
                    cmp(src, o_vmem, p + (p & ~jm), j, True)

            else:
                h = k // 2
                hm = h - 1

                @plsc.parallel_loop(0, _N // 4, unroll=4)
                def _(q):
                    p = q + (q & ~hm)
                    cmp(src, o_vmem, p + (p & ~jm), j, True)

                @plsc.parallel_loop(0, _N // 4, unroll=4)
                def _(q):
                    p = q + (q & ~hm) + h
                    cmp(src, o_vmem, p + (p & ~jm), j, False)

            first = False
            j //= 2
        k *= 2


def _sc_sort(c):
    nsc = c.shape[0]

    @pl.kernel(
        out_type=jax.ShapeDtypeStruct(c.shape, c.dtype),
        mesh=plsc.VectorSubcoreMesh(core_axis_name="c", subcore_axis_name="s"),
        compiler_params=pltpu.CompilerParams(use_tc_tiling_on_sc=False),
    )
    def sc_fn(c_hbm, o_hbm):
        pltpu.emit_pipeline(
            _sc_tile_body,
            grid=(nsc, _N // 16),
            in_specs=[pl.BlockSpec((1, _N, 16), lambda b, t: (b, 0, t))],
            out_specs=[pl.BlockSpec((1, _N, 16), lambda b, t: (b, 0, t))],
            core_axis_name=("c", "s"),
            dimension_semantics=(pltpu.PARALLEL, pltpu.PARALLEL),
        )(c_hbm, o_hbm)

    return sc_fn(c)


def kernel(x):
    n_bsize, n_feats, n_cols, n_rows = x.shape
    x3 = x.reshape(n_bsize, n_feats, n_cols * n_rows)
    c_sc = pl.pallas_call(
        _corr_only_body,
        grid=(1,),
        in_specs=[pl.BlockSpec((_NSC, n_feats, _N), lambda b: (0, 0, 0))],
        out_specs=pl.BlockSpec((_NSC, _N, _N), lambda b: (0, 0, 0)),
        out_shape=jax.ShapeDtypeStruct((_NSC, _N, _N), jnp.float32),
    )(x3[:_NSC])
    out_sc = _sc_sort(c_sc)
    out_tc = pl.pallas_call(
        _corr_sort_body,
        grid=((n_bsize - _NSC) // _BPS,),
        in_specs=[pl.BlockSpec((_BPS, n_feats, _N), lambda b: (b, 0, 0))],
        out_specs=pl.BlockSpec((_BPS, _N, _N), lambda b: (b, 0, 0)),
        out_shape=jax.ShapeDtypeStruct((n_bsize - _NSC, _N, _N), jnp.float32),
    )(x3[_NSC:])
    out = jnp.concatenate([out_sc, out_tc], axis=0)
    return out.reshape(n_bsize, _N, n_cols, n_rows)


# speedup vs baseline: 1.1808x; 1.0026x over previous
import jax
import jax.numpy as jnp
from jax.experimental import pallas as pl
from jax.experimental.pallas import tpu as pltpu
from jax.experimental.pallas import tpu_sc as plsc

_N = 256
_FEATS = 768
_BATCH = 32

_PJ = {1: 32, 2: 64, 4: 128, 8: 8, 16: 16, 32: 1, 64: 2, 128: 4}
_DK = {2: 64, 4: 128, 8: 8, 16: 16, 32: 1, 64: 2, 128: 4}


def _stage(a, k, j):
    n, cols = a.shape
    pj = _PJ[j]
    dk = _DK.get(k)
    if pj >= 8:
        g = n // (2 * pj)
        a4 = a.reshape(g, 2, pj, cols)
        mn = jnp.minimum(a4[:, 0], a4[:, 1]).reshape(n // 2, cols)
        mx = jnp.maximum(a4[:, 0], a4[:, 1]).reshape(n // 2, cols)
        if dk is None:
            nl, nh = mx, mn
        else:
            d = dk // 2 if dk >= 2 * pj else dk
            if d >= 8:
                m7 = mn.reshape(n // (4 * d), 2, d, cols)
                x7 = mx.reshape(n // (4 * d), 2, d, cols)
                nl = jnp.concatenate([x7[:, :1], m7[:, 1:]], axis=1)
                nl = nl.reshape(n // 2, cols)
                nh = jnp.concatenate([m7[:, :1], x7[:, 1:]], axis=1)
                nh = nh.reshape(n // 2, cols)
            else:
                q = jax.lax.broadcasted_iota(jnp.int32, (n // 2, cols), 0)
                ascm = (q & d) != 0
                nl = jnp.where(ascm, mn, mx)
                nh = jnp.where(ascm, mx, mn)
        return jnp.stack(
            [nl.reshape(g, pj, cols), nh.reshape(g, pj, cols)], axis=1
        ).reshape(n, cols)
    row = jax.lax.broadcasted_iota(jnp.int32, (n, cols), 0)
    bitp = (row & pj) != 0
    if pj == 4:
        p = jnp.roll(a.reshape(n // 8, 8, cols), 4, axis=1).reshape(n, cols)
    else:
        p = jnp.where(bitp, jnp.roll(a, pj, axis=0), jnp.roll(a, -pj, axis=0))
    if dk is None:
        take_min = bitp
    else:
        take_min = jnp.logical_xor((row & dk) != 0, bitp)
    return jnp.where(take_min, jnp.minimum(a, p), jnp.maximum(a, p))


_BPS = 2
_CW = 128
_NSC = 4


def _corr_sort_body(x_ref, o_ref):
    for b in range(_BPS):
        x = x_ref[b]
        for h in range(_N // _CW):
            cs = slice(h * _CW, (h + 1) * _CW)
            a = jax.lax.dot_general(
                x,
                x[:, cs],
                (((0,), (0,)), ((), ())),
                preferred_element_type=jnp.float32,
            ) * (1.0 / _FEATS)

            k = 2
            while k <= _N:
                j = k // 2
                while j >= 1:
                    a = _stage(a, k, j)
                    j //= 2
                k *= 2
            o_ref[b, :, cs] = (
                a.reshape(8, 4, 8, _CW).transpose(2, 1, 0, 3).reshape(_N, _CW)
            )


def _corr_only_body(x_ref, o_ref):
    for b in range(_NSC):
        x = x_ref[b]
        o_ref[b] = jax.lax.dot_general(
            x, x, (((0,), (0,)), ((), ())), preferred_element_type=jnp.float32
        ) * (1.0 / _FEATS)


def _sc_tile_body(c_vmem, o_vmem):
    def cmp(src, dst, i, j, desc):
        lo = src[0, i]
        hi = src[0, i + j]
        if desc:
            dst[0, i] = jnp.maximum(lo, hi)
            dst[0, i + j] = jnp.minimum(lo, hi)
        else:
            dst[0, i] = jnp.minimum(lo, hi)
            dst[0, i + j] = jnp.maximum(lo, hi)

    first = True
    k = 2
    while k <= _N:
        j = k // 2
        while j >= 1:
            src = c_vmem if first else o_vmem
            jm = j - 1
            if k == _N:

                @plsc.parallel_loop(0, _N // 2, unroll=4)
                def _(p):
                    cmp(src, o_vmem, p + (p & ~jm), j, True)

            else:
                h = k // 2
                hm = h - 1

                @plsc.parallel_loop(0, _N // 4, unroll=4)
                def _(q):
                    p = q + (q & ~hm)
                    cmp(src, o_vmem, p + (p & ~jm), j, True)

                @plsc.parallel_loop(0, _N // 4, unroll=4)
                def _(q):
                    p = q + (q & ~hm) + h
                    cmp(src, o_vmem, p + (p & ~jm), j, False)

            first = False
            j //= 2
        k *= 2


def _sc_sort(c):
    nsc = c.shape[0]

    @pl.kernel(
        out_type=jax.ShapeDtypeStruct(c.shape, c.dtype),
        mesh=plsc.VectorSubcoreMesh(core_axis_name="c", subcore_axis_name="s"),
        compiler_params=pltpu.CompilerParams(use_tc_tiling_on_sc=False),
    )
    def sc_fn(c_hbm, o_hbm):
        pltpu.emit_pipeline(
            _sc_tile_body,
            grid=(nsc, _N // 16),
            in_specs=[pl.BlockSpec((1, _N, 16), lambda b, t: (b, 0, t))],
            out_specs=[pl.BlockSpec((1, _N, 16), lambda b, t: (b, 0, t))],
            core_axis_name=("c", "s"),
            dimension_semantics=(pltpu.PARALLEL, pltpu.PARALLEL),
        )(c_hbm, o_hbm)

    return sc_fn(c)


def kernel(x):
    n_bsize, n_feats, n_cols, n_rows = x.shape
    x3 = x.reshape(n_bsize, n_feats, n_cols * n_rows)
    c_sc = pl.pallas_call(
        _corr_only_body,
        grid=(1,),
        in_specs=[pl.BlockSpec((_NSC, n_feats, _N), lambda b: (0, 0, 0))],
        out_specs=pl.BlockSpec((_NSC, _N, _N), lambda b: (0, 0, 0)),
        out_shape=jax.ShapeDtypeStruct((_NSC, _N, _N), jnp.float32),
    )(x3[:_NSC])
    out_tc = pl.pallas_call(
        _corr_sort_body,
        grid=((n_bsize - _NSC) // _BPS,),
        in_specs=[pl.BlockSpec((_BPS, n_feats, _N), lambda b: (b, 0, 0))],
        out_specs=pl.BlockSpec((_BPS, _N, _N), lambda b: (b, 0, 0)),
        out_shape=jax.ShapeDtypeStruct((n_bsize - _NSC, _N, _N), jnp.float32),
    )(x3[_NSC:])
    out_sc = _sc_sort(c_sc)
    out = jnp.concatenate([out_sc, out_tc], axis=0)
    return out.reshape(n_bsize, _N, n_cols, n_rows)


# speedup vs baseline: 1.8410x; 1.5590x over previous
import jax
import jax.numpy as jnp
from jax.experimental import pallas as pl
from jax.experimental.pallas import tpu as pltpu

_N = 256
_FEATS = 768
_BATCH = 32

_PJ = {1: 32, 2: 64, 4: 128, 8: 8, 16: 16, 32: 1, 64: 2, 128: 4}
_DK = {2: 64, 4: 128, 8: 8, 16: 16, 32: 1, 64: 2, 128: 4}


def _stage(a, k, j):
    n, cols = a.shape
    pj = _PJ[j]
    dk = _DK.get(k)
    if pj >= 8:
        g = n // (2 * pj)
        a4 = a.reshape(g, 2, pj, cols)
        mn = jnp.minimum(a4[:, 0], a4[:, 1]).reshape(n // 2, cols)
        mx = jnp.maximum(a4[:, 0], a4[:, 1]).reshape(n // 2, cols)
        if dk is None:
            nl, nh = mx, mn
        else:
            d = dk // 2 if dk >= 2 * pj else dk
            if d >= 8:
                m7 = mn.reshape(n // (4 * d), 2, d, cols)
                x7 = mx.reshape(n // (4 * d), 2, d, cols)
                nl = jnp.concatenate([x7[:, :1], m7[:, 1:]], axis=1)
                nl = nl.reshape(n // 2, cols)
                nh = jnp.concatenate([m7[:, :1], x7[:, 1:]], axis=1)
                nh = nh.reshape(n // 2, cols)
            else:
                q = jax.lax.broadcasted_iota(jnp.int32, (n // 2, cols), 0)
                ascm = (q & d) != 0
                nl = jnp.where(ascm, mn, mx)
                nh = jnp.where(ascm, mx, mn)
        return jnp.stack(
            [nl.reshape(g, pj, cols), nh.reshape(g, pj, cols)], axis=1
        ).reshape(n, cols)
    row = jax.lax.broadcasted_iota(jnp.int32, (n, cols), 0)
    bitp = (row & pj) != 0
    if pj == 4:
        p = jnp.roll(a.reshape(n // 8, 8, cols), 4, axis=1).reshape(n, cols)
    else:
        p = jnp.where(bitp, jnp.roll(a, pj, axis=0), jnp.roll(a, -pj, axis=0))
    if dk is None:
        take_min = bitp
    else:
        take_min = jnp.logical_xor((row & dk) != 0, bitp)
    return jnp.where(take_min, jnp.minimum(a, p), jnp.maximum(a, p))


_BPS = 4
_CW = 128


def _corr_sort_body(x_ref, o_ref):
    for b in range(_BPS):
        x = x_ref[b].astype(jnp.bfloat16)
        for h in range(_N // _CW):
            cs = slice(h * _CW, (h + 1) * _CW)
            a = jax.lax.dot_general(
                x,
                x[:, cs],
                (((0,), (0,)), ((), ())),
                preferred_element_type=jnp.float32,
            ) * (1.0 / _FEATS)

            k = 2
            while k <= _N:
                j = k // 2
                while j >= 1:
                    a = _stage(a, k, j)
                    j //= 2
                k *= 2
            o_ref[b, :, cs] = (
                a.reshape(8, 4, 8, _CW).transpose(2, 1, 0, 3).reshape(_N, _CW)
            )


def kernel(x):
    n_bsize, n_feats, n_cols, n_rows = x.shape
    x3 = x.reshape(n_bsize, n_feats, n_cols * n_rows)
    out = pl.pallas_call(
        _corr_sort_body,
        grid=(n_bsize // _BPS,),
        in_specs=[pl.BlockSpec((_BPS, n_feats, _N), lambda b: (b, 0, 0))],
        out_specs=pl.BlockSpec((_BPS, _N, _N), lambda b: (b, 0, 0)),
        out_shape=jax.ShapeDtypeStruct((n_bsize, _N, _N), jnp.float32),
    )(x3)
    return out.reshape(n_bsize, _N, n_cols, n_rows)
